# Initial kernel scaffold; baseline (speedup 1.0000x reference)
import jax
import jax.numpy as jnp
from jax.experimental import pallas as pl


def kernel(score, target):
    raise NotImplementedError("write your pallas kernel here")


# speedup vs baseline: 2.5559x; 2.5559x over previous
import functools

import numpy as np

import jax
import jax.numpy as jnp
from jax.experimental import pallas as pl
from jax.experimental.pallas import tpu as pltpu

_IGNORE = -1
_VMEM_LIMIT = 48 * 1024 * 1024


def _upsample_matrix(src, dst, dst_pad):
    m = np.zeros((dst_pad, src), np.float32)
    d = np.arange(dst)
    s = np.maximum((d + 0.5) * (src / dst) - 0.5, 0.0)
    i0 = np.minimum(np.floor(s), src - 1).astype(np.int64)
    i1 = np.minimum(i0 + 1, src - 1)
    w1 = (s - i0).astype(np.float32)
    m[d, i0] += 1.0 - w1
    m[d, i1] += w1
    return m


def _ce_body(score_ref, wy_ref, wx_ref, lbl_ref, sum_ref, cnt_ref, xs_ref, *,
             num_classes):
    wyb = wy_ref[...]
    wx = wx_ref[...]
    t = lbl_ref[0]

    m = None
    picked = None
    for cc in range(num_classes):
        ch = score_ref[0, cc].astype(jnp.bfloat16)
        yc = jnp.dot(wyb, ch, preferred_element_type=jnp.float32)
        xc = jnp.dot(yc.astype(jnp.bfloat16), wx,
                     preferred_element_type=jnp.float32)
        xs_ref[cc] = xc
        hit = jnp.where(t == cc, xc, 0.0)
        if cc == 0:
            m, picked = xc, hit
        else:
            m = jnp.maximum(m, xc)
            picked = picked + hit

    s = None
    for cc in range(num_classes):
        e = jnp.exp(xs_ref[cc] - m)
        s = e if s is None else s + e

    loss = m + jnp.log(s) - picked
    valid = t != _IGNORE
    sum_ref[0] = jnp.sum(jnp.where(valid, loss, 0.0), axis=0, keepdims=True)
    cnt_ref[0] = jnp.sum(valid.astype(jnp.float32), axis=0, keepdims=True)


def kernel(score, target):
    n, c, hs, ws = score.shape
    _, h, w = target.shape

    rb = min(h, 256)
    bands = pl.cdiv(h, rb)
    h_pad = bands * rb
    if h_pad != h:
        target = jnp.pad(target, ((0, 0), (0, h_pad - h), (0, 0)),
                         constant_values=_IGNORE)

    wy = jnp.asarray(_upsample_matrix(hs, h, h_pad)).astype(jnp.bfloat16)
    wx = jnp.asarray(_upsample_matrix(ws, w, w).T).astype(jnp.bfloat16)
    work = n * bands

    body = functools.partial(_ce_body, num_classes=c)
    part_sum, part_cnt = pl.pallas_call(
        body,
        out_shape=(jax.ShapeDtypeStruct((work, 1, w), jnp.float32),
                   jax.ShapeDtypeStruct((work, 1, w), jnp.float32)),
        grid_spec=pltpu.PrefetchScalarGridSpec(
            num_scalar_prefetch=0,
            grid=(work,),
            in_specs=[
                pl.BlockSpec((1, c, hs, ws), lambda i: (i // bands, 0, 0, 0)),
                pl.BlockSpec((rb, hs), lambda i: (i % bands, 0)),
                pl.BlockSpec((ws, w), lambda i: (0, 0)),
                pl.BlockSpec((1, rb, w), lambda i: (i // bands, i % bands, 0)),
            ],
            out_specs=[
                pl.BlockSpec((1, 1, w), lambda i: (i, 0, 0)),
                pl.BlockSpec((1, 1, w), lambda i: (i, 0, 0)),
            ],
            scratch_shapes=[pltpu.VMEM((c, rb, w), jnp.float32)],
        ),
        compiler_params=pltpu.CompilerParams(
            dimension_semantics=("parallel",),
            vmem_limit_bytes=_VMEM_LIMIT),
    )(score, wy, wx, target)

    return (jnp.sum(part_sum) / jnp.sum(part_cnt)).astype(jnp.float32)


# speedup vs baseline: 3.0660x; 1.1996x over previous
import functools

import numpy as np

import jax
import jax.numpy as jnp
from jax.experimental import pallas as pl
from jax.experimental.pallas import tpu as pltpu

_IGNORE = -1
_VMEM_LIMIT = 48 * 1024 * 1024


def _upsample_matrix(src, dst, dst_pad):
    m = np.zeros((dst_pad, src), np.float32)
    d = np.arange(dst)
    s = np.maximum((d + 0.5) * (src / dst) - 0.5, 0.0)
    i0 = np.minimum(np.floor(s), src - 1).astype(np.int64)
    i1 = np.minimum(i0 + 1, src - 1)
    w1 = (s - i0).astype(np.float32)
    m[d, i0] += 1.0 - w1
    m[d, i1] += w1
    return m


def _ce_body(score_ref, wy_ref, wx_ref, lbl_ref, sum_ref, cnt_ref, xs_ref, *,
             num_classes, chunk):
    wyb = wy_ref[...]
    wx = wx_ref[...]
    rb = wy_ref.shape[0]

    for cc in range(num_classes):
        ch = score_ref[0, cc].astype(jnp.bfloat16)
        yc = jnp.dot(wyb, ch, preferred_element_type=jnp.float32)
        xs_ref[cc] = jnp.dot(yc.astype(jnp.bfloat16), wx,
                             preferred_element_type=jnp.float32)

    lsum = None
    lcnt = None
    for r0 in range(0, rb, chunk):
        t = lbl_ref[0, r0:r0 + chunk, :]
        m = None
        for cc in range(num_classes):
            x = xs_ref[cc, r0:r0 + chunk, :]
            m = x if m is None else jnp.maximum(m, x)
        s = None
        picked = None
        for cc in range(num_classes):
            x = xs_ref[cc, r0:r0 + chunk, :]
            e = jnp.exp(x - m)
            s = e if s is None else s + e
            hit = jnp.where(t == cc, x, 0.0)
            picked = hit if picked is None else picked + hit
        loss = m + jnp.log(s) - picked
        valid = t != _IGNORE
        ls = jnp.sum(jnp.where(valid, loss, 0.0), axis=0, keepdims=True)
        lc = jnp.sum(valid.astype(jnp.float32), axis=0, keepdims=True)
        lsum = ls if lsum is None else lsum + ls
        lcnt = lc if lcnt is None else lcnt + lc
    sum_ref[0] = lsum
    cnt_ref[0] = lcnt


def kernel(score, target):
    n, c, hs, ws = score.shape
    _, h, w = target.shape

    rb = min(h, 512)
    bands = pl.cdiv(h, rb)
    h_pad = bands * rb
    if h_pad != h:
        target = jnp.pad(target, ((0, 0), (0, h_pad - h), (0, 0)),
                         constant_values=_IGNORE)

    wy = jnp.asarray(_upsample_matrix(hs, h, h_pad)).astype(jnp.bfloat16)
    wx = jnp.asarray(_upsample_matrix(ws, w, w).T).astype(jnp.bfloat16)
    work = n * bands

    chunk = 128 if rb % 128 == 0 else rb

    body = functools.partial(_ce_body, num_classes=c, chunk=chunk)
    part_sum, part_cnt = pl.pallas_call(
        body,
        out_shape=(jax.ShapeDtypeStruct((work, 1, w), jnp.float32),
                   jax.ShapeDtypeStruct((work, 1, w), jnp.float32)),
        grid_spec=pltpu.PrefetchScalarGridSpec(
            num_scalar_prefetch=0,
            grid=(work,),
            in_specs=[
                pl.BlockSpec((1, c, hs, ws), lambda i: (i // bands, 0, 0, 0)),
                pl.BlockSpec((rb, hs), lambda i: (i % bands, 0)),
                pl.BlockSpec((ws, w), lambda i: (0, 0)),
                pl.BlockSpec((1, rb, w), lambda i: (i // bands, i % bands, 0)),
            ],
            out_specs=[
                pl.BlockSpec((1, 1, w), lambda i: (i, 0, 0)),
                pl.BlockSpec((1, 1, w), lambda i: (i, 0, 0)),
            ],
            scratch_shapes=[pltpu.VMEM((c, rb, w), jnp.float32)],
        ),
        compiler_params=pltpu.CompilerParams(
            dimension_semantics=("parallel",),
            vmem_limit_bytes=_VMEM_LIMIT),
    )(score, wy, wx, target)

    return (jnp.sum(part_sum) / jnp.sum(part_cnt)).astype(jnp.float32)


# speedup vs baseline: 3.5731x; 1.1654x over previous
import functools

import numpy as np

import jax
import jax.numpy as jnp
from jax.experimental import pallas as pl
from jax.experimental.pallas import tpu as pltpu

_IGNORE = -1
_VMEM_LIMIT = 48 * 1024 * 1024


def _upsample_matrix(src, dst, dst_pad):
    m = np.zeros((dst_pad, src), np.float32)
    d = np.arange(dst)
    s = np.maximum((d + 0.5) * (src / dst) - 0.5, 0.0)
    i0 = np.minimum(np.floor(s), src - 1).astype(np.int64)
    i1 = np.minimum(i0 + 1, src - 1)
    w1 = (s - i0).astype(np.float32)
    m[d, i0] += 1.0 - w1
    m[d, i1] += w1
    return m


def _ce_body(score_ref, wy_ref, wx_ref, lbl_ref, sum_ref, cnt_ref,
             ycs_ref, xs_ref, m_ref, *, num_classes, chunk, sub):
    wx = wx_ref[...]
    rb = wy_ref.shape[0]

    for cc in range(num_classes):
        ch = score_ref[0, cc].astype(jnp.bfloat16)
        yc = jnp.dot(wy_ref[...], ch, preferred_element_type=jnp.float32)
        ycs_ref[cc] = yc.astype(jnp.bfloat16)

    lsum = None
    lcnt = None
    for r0 in range(0, rb, chunk):
        m = None
        for cc in range(num_classes):
            xc = jnp.dot(ycs_ref[cc, r0:r0 + chunk, :], wx,
                         preferred_element_type=jnp.float32)
            xs_ref[cc] = xc
            m = xc if m is None else jnp.maximum(m, xc)
        m_ref[...] = m

        for s0 in range(0, chunk, sub):
            t = lbl_ref[0, r0 + s0:r0 + s0 + sub, :]
            ms = m_ref[s0:s0 + sub, :]
            s = None
            picked = None
            for cc in range(num_classes):
                x = xs_ref[cc, s0:s0 + sub, :]
                e = jnp.exp(x - ms)
                s = e if s is None else s + e
                hit = jnp.where(t == cc, x, 0.0)
                picked = hit if picked is None else picked + hit
            loss = ms + jnp.log(s) - picked
            valid = t != _IGNORE
            ls = jnp.sum(jnp.where(valid, loss, 0.0), axis=0, keepdims=True)
            lc = jnp.sum(valid.astype(jnp.float32), axis=0, keepdims=True)
            lsum = ls if lsum is None else lsum + ls
            lcnt = lc if lcnt is None else lcnt + lc
    sum_ref[0] = lsum
    cnt_ref[0] = lcnt


def kernel(score, target):
    n, c, hs, ws = score.shape
    _, h, w = target.shape

    rb = min(h, 512)
    bands = pl.cdiv(h, rb)
    h_pad = bands * rb
    if h_pad != h:
        target = jnp.pad(target, ((0, 0), (0, h_pad - h), (0, 0)),
                         constant_values=_IGNORE)

    wy = jnp.asarray(_upsample_matrix(hs, h, h_pad)).astype(jnp.bfloat16)
    wx = jnp.asarray(_upsample_matrix(ws, w, w).T).astype(jnp.bfloat16)
    work = n * bands

    chunk = 64 if rb % 64 == 0 else rb
    sub = 16 if chunk % 16 == 0 else chunk

    body = functools.partial(_ce_body, num_classes=c, chunk=chunk, sub=sub)
    part_sum, part_cnt = pl.pallas_call(
        body,
        out_shape=(jax.ShapeDtypeStruct((work, 1, w), jnp.float32),
                   jax.ShapeDtypeStruct((work, 1, w), jnp.float32)),
        grid_spec=pltpu.PrefetchScalarGridSpec(
            num_scalar_prefetch=0,
            grid=(work,),
            in_specs=[
                pl.BlockSpec((1, c, hs, ws), lambda i: (i // bands, 0, 0, 0)),
                pl.BlockSpec((rb, hs), lambda i: (i % bands, 0)),
                pl.BlockSpec((ws, w), lambda i: (0, 0)),
                pl.BlockSpec((1, rb, w), lambda i: (i // bands, i % bands, 0)),
            ],
            out_specs=[
                pl.BlockSpec((1, 1, w), lambda i: (i, 0, 0)),
                pl.BlockSpec((1, 1, w), lambda i: (i, 0, 0)),
            ],
            scratch_shapes=[pltpu.VMEM((c, rb, hs), jnp.bfloat16),
                            pltpu.VMEM((c, chunk, w), jnp.float32),
                            pltpu.VMEM((chunk, w), jnp.float32)],
        ),
        compiler_params=pltpu.CompilerParams(
            dimension_semantics=("parallel",),
            vmem_limit_bytes=_VMEM_LIMIT),
    )(score, wy, wx, target)

    return (jnp.sum(part_sum) / jnp.sum(part_cnt)).astype(jnp.float32)
